# baseline (speedup 1.0000x reference)
import functools

import numpy as np
import jax
import jax.numpy as jnp
from jax.experimental import pallas as pl
from jax.experimental.pallas import tpu as pltpu


def _bilinear_matrix_np(n_in, n_out):
    src = np.arange(n_out, dtype=np.float32) * np.float32(
        (n_in - 1) / (n_out - 1))
    i0 = np.clip(np.floor(src).astype(np.int32), 0, n_in - 2)
    w1 = (src - i0.astype(np.float32)).astype(np.float32)
    A = np.zeros((n_out, n_in), np.float32)
    A[np.arange(n_out), i0] += np.float32(1.0) - w1
    A[np.arange(n_out), i0 + 1] += w1
    return A


@functools.lru_cache(maxsize=None)
def _upsample_mats_np(H, W):
    H2, W2 = 2 * H, 2 * W
    ah = _bilinear_matrix_np(H, H2)
    aw = _bilinear_matrix_np(W, W2)
    ahp = np.zeros((H2 + 2, H), np.float32)
    ahp[1:H2 + 1] = ah
    awp = np.zeros((W2 + 2, W), np.float32)
    awp[1:W2 + 1] = aw
    m = np.concatenate(
        [np.kron(ahp, awp[kx:kx + W2, :]) for kx in range(3)], axis=0)
    return m


def _upconv_kernel(x_ref, m_ref, w_ref, b_ref, o_ref, u_ref, acc_ref,
                   *, W2, HW2, ROWS):
    C = x_ref.shape[1]
    OC = o_ref.shape[2]
    xx = jnp.concatenate([x_ref[0], x_ref[1]],
                         axis=0).astype(jnp.bfloat16)
    u_ref[...] = jax.lax.dot_general(
        m_ref[...], xx, dimension_numbers=(((1,), (1,)), ((), ())),
        preferred_element_type=jnp.float32).astype(jnp.bfloat16)
    for s in range(2):
        for kx in range(3):
            base = kx * ROWS
            for ky in range(3):
                patch = u_ref[base + ky * W2: base + ky * W2 + HW2,
                              s * C:(s + 1) * C]
                d = jnp.dot(patch, w_ref[3 * ky + kx],
                            preferred_element_type=jnp.float32)
                if kx == 0 and ky == 0:
                    acc_ref[...] = d + b_ref[...]
                else:
                    acc_ref[...] += d
        o_ref[s] = jnp.maximum(acc_ref[...], 0.0).astype(jnp.bfloat16)


def _upconv_conv_tanh_kernel(x_ref, a_ref, s_ref, w_ref, b_ref, wp_ref,
                             b3_ref, o_ref, vh_ref, u_ref, acc_ref, p_ref,
                             zi_ref, z_ref, *, H2p, W, C, W2, HW2, ROWS):
    OC = acc_ref.shape[1]
    _sep_upconv_relu(x_ref, a_ref, s_ref, w_ref, b_ref, vh_ref, u_ref,
                     acc_ref, H2p=H2p, W=W, C=C, W2=W2, HW2=HW2, ROWS=ROWS)
    p_ref[0:128, :] = jnp.zeros((128, OC), jnp.bfloat16)
    p_ref[128:128 + HW2, :] = acc_ref[...].astype(jnp.bfloat16)
    p_ref[128 + HW2:, :] = jnp.zeros((128, OC), jnp.bfloat16)
    ncols = jax.lax.broadcasted_iota(jnp.int32, (ROWS, OC), 0) % W2
    for kx in range(3):
        start = 128 - W2 - 1 + kx
        shifted = p_ref[start: start + ROWS, :]
        if kx == 0:
            shifted = jnp.where(ncols == 0, jnp.bfloat16(0), shifted)
        elif kx == 2:
            shifted = jnp.where(ncols == W2 - 1, jnp.bfloat16(0), shifted)
        zi_ref[:, kx * OC:(kx + 1) * OC] = shifted
    z_ref[...] = jnp.dot(zi_ref[...], wp_ref[...],
                         preferred_element_type=jnp.float32)
    o3 = jnp.broadcast_to(b3_ref[...], (HW2, 128))
    for ky in range(3):
        o3 = o3 + z_ref[ky * W2: ky * W2 + HW2, 128 * ky: 128 * ky + 128]
    o_ref[0] = jnp.tanh(jnp.transpose(o3[:, 0:3], (1, 0)))


def _sep_upconv_relu(x_ref, a_ref, s_ref, w_ref, b_ref, vh_ref, u_ref,
                     acc_ref, *, H2p, W, C, W2, HW2, ROWS):
    OC = acc_ref.shape[1]
    vh2 = jnp.dot(a_ref[...], x_ref[0],
                  preferred_element_type=jnp.float32)
    vh_ref[...] = jnp.reshape(vh2, (H2p, W, C)).astype(jnp.bfloat16)
    for kx in range(3):
        u3 = jax.lax.dot_general(
            s_ref[kx], vh_ref[...],
            dimension_numbers=(((2,), (1,)), ((0,), (0,))),
            preferred_element_type=jnp.float32)
        u_ref[kx * ROWS:(kx + 1) * ROWS, :] = (
            jnp.reshape(u3, (ROWS, C)).astype(jnp.bfloat16))
    for kx in range(3):
        base = kx * ROWS
        for ky in range(3):
            patch = u_ref[base + ky * W2: base + ky * W2 + HW2, :]
            d = jnp.dot(patch, w_ref[3 * ky + kx],
                        preferred_element_type=jnp.float32)
            if kx == 0 and ky == 0:
                acc_ref[...] = d + b_ref[...]
            else:
                acc_ref[...] += d
    acc_ref[...] = jnp.maximum(acc_ref[...], 0.0)


def _up_conv(x_cm, H, W, weight, bias):
    N, C, HW = x_cm.shape
    OC = weight.shape[-1]
    H2, W2 = 2 * H, 2 * W
    HW2 = H2 * W2
    ROWS = (H2 + 2) * W2

    m = jnp.asarray(_upsample_mats_np(H, W)).astype(jnp.bfloat16)
    w_taps = weight.reshape(9, C, OC).astype(jnp.bfloat16)
    b_row = bias.reshape(1, OC)

    _kfn = functools.partial(_upconv_kernel, W2=W2, HW2=HW2, ROWS=ROWS)
    return pl.pallas_call(
        _kfn,
        out_shape=jax.ShapeDtypeStruct((N, HW2, OC), jnp.bfloat16),
        grid=(N // 2,),
        in_specs=[
            pl.BlockSpec((2, C, HW), lambda n: (n, 0, 0)),
            pl.BlockSpec((3 * ROWS, HW), lambda n: (0, 0)),
            pl.BlockSpec((9, C, OC), lambda n: (0, 0, 0)),
            pl.BlockSpec((1, OC), lambda n: (0, 0)),
        ],
        out_specs=pl.BlockSpec((2, HW2, OC), lambda n: (n, 0, 0)),
        scratch_shapes=[pltpu.VMEM((3 * ROWS, 2 * C), jnp.bfloat16),
                        pltpu.VMEM((HW2, OC), jnp.float32)],
        compiler_params=pltpu.CompilerParams(
            dimension_semantics=("parallel",),
            vmem_limit_bytes=100 * 1024 * 1024,
        ),
    )(x_cm, m, w_taps, b_row)


def _up_conv_tanh_fused(x_flat, H, W, w2, b2, w3, b3):
    N, HW, C = x_flat.shape
    OC = w2.shape[-1]
    OC3 = w3.shape[-1]
    H2, W2 = 2 * H, 2 * W
    H2p = H2 + 2
    HW2 = H2 * W2
    ROWS = H2p * W2

    ah = _bilinear_matrix_np(H, H2)
    ahp = np.zeros((H2p, H), np.float32)
    ahp[1:H2 + 1] = ah
    aw = _bilinear_matrix_np(W, W2)
    awp = np.zeros((W2 + 2, W), np.float32)
    awp[1:W2 + 1] = aw
    s_np = np.stack([np.broadcast_to(awp[kx:kx + W2], (H2p, W2, W))
                     for kx in range(3)])

    a_mat = jnp.asarray(ahp).astype(jnp.bfloat16)
    s_mat = jnp.asarray(s_np).astype(jnp.bfloat16)
    w_taps = w2.reshape(9, C, OC).astype(jnp.bfloat16)
    b_row = b2.reshape(1, OC)
    wp = jnp.concatenate(
        [jnp.pad(w3[ky].reshape(3 * OC, OC3), ((0, 0), (0, 128 - OC3)))
         for ky in range(3)], axis=-1).astype(jnp.bfloat16)
    b3_row = jnp.pad(b3.reshape(1, OC3), ((0, 0), (0, 128 - OC3)))

    _kfn = functools.partial(_upconv_conv_tanh_kernel, H2p=H2p, W=W, C=C,
                             W2=W2, HW2=HW2, ROWS=ROWS)
    return pl.pallas_call(
        _kfn,
        out_shape=jax.ShapeDtypeStruct((N, OC3, HW2), jnp.float32),
        grid=(N,),
        in_specs=[
            pl.BlockSpec((1, H, W * C), lambda n: (n, 0, 0)),
            pl.BlockSpec((H2p, H), lambda n: (0, 0)),
            pl.BlockSpec((3, H2p, W2, W), lambda n: (0, 0, 0, 0)),
            pl.BlockSpec((9, C, OC), lambda n: (0, 0, 0)),
            pl.BlockSpec((1, OC), lambda n: (0, 0)),
            pl.BlockSpec((3 * OC, 384), lambda n: (0, 0)),
            pl.BlockSpec((1, 128), lambda n: (0, 0)),
        ],
        out_specs=pl.BlockSpec((1, OC3, HW2), lambda n: (n, 0, 0)),
        scratch_shapes=[pltpu.VMEM((H2p, W, C), jnp.bfloat16),
                        pltpu.VMEM((3 * ROWS, C), jnp.bfloat16),
                        pltpu.VMEM((HW2, OC), jnp.float32),
                        pltpu.VMEM((HW2 + 256, OC), jnp.bfloat16),
                        pltpu.VMEM((ROWS, 3 * OC), jnp.bfloat16),
                        pltpu.VMEM((ROWS, 384), jnp.float32)],
        compiler_params=pltpu.CompilerParams(
            dimension_semantics=("parallel",),
            vmem_limit_bytes=100 * 1024 * 1024,
        ),
    )(x_flat.reshape(N, H, W * C), a_mat, s_mat, w_taps, b_row, wp, b3_row)


def kernel(x, w1, b1, w2, b2, w3, b3):
    N, C, H, W = x.shape
    y = _up_conv(x.reshape(N, C, H * W), H, W, w1, b1)
    y = _up_conv_tanh_fused(y, 2 * H, 2 * W, w2, b2, w3, b3)
    return y.reshape(N, 3, 4 * H, 4 * W)


# speedup vs baseline: 1.0145x; 1.0145x over previous
import functools

import numpy as np
import jax
import jax.numpy as jnp
from jax.experimental import pallas as pl
from jax.experimental.pallas import tpu as pltpu


def _bilinear_matrix_np(n_in, n_out):
    src = np.arange(n_out, dtype=np.float32) * np.float32(
        (n_in - 1) / (n_out - 1))
    i0 = np.clip(np.floor(src).astype(np.int32), 0, n_in - 2)
    w1 = (src - i0.astype(np.float32)).astype(np.float32)
    A = np.zeros((n_out, n_in), np.float32)
    A[np.arange(n_out), i0] += np.float32(1.0) - w1
    A[np.arange(n_out), i0 + 1] += w1
    return A


@functools.lru_cache(maxsize=None)
def _upsample_mats_np(H, W):
    H2, W2 = 2 * H, 2 * W
    ah = _bilinear_matrix_np(H, H2)
    aw = _bilinear_matrix_np(W, W2)
    ahp = np.zeros((H2 + 2, H), np.float32)
    ahp[1:H2 + 1] = ah
    awp = np.zeros((W2 + 2, W), np.float32)
    awp[1:W2 + 1] = aw
    m = np.concatenate(
        [np.kron(ahp, awp[kx:kx + W2, :]) for kx in range(3)], axis=0)
    return m


def _upconv_kernel(x_ref, m_ref, w_ref, b_ref, o_ref, u_ref, acc_ref,
                   *, W2, HW2, ROWS):
    C = x_ref.shape[2]
    OC = o_ref.shape[2]
    xx = jnp.concatenate([x_ref[0], x_ref[1]], axis=1)
    u_ref[...] = jnp.dot(m_ref[...], xx,
                         preferred_element_type=jnp.float32).astype(jnp.bfloat16)
    for s in range(2):
        acc_ref[...] = jnp.broadcast_to(b_ref[...], (HW2, OC))
        for kx in range(3):
            base = kx * ROWS
            for ky in range(3):
                patch = u_ref[base + ky * W2: base + ky * W2 + HW2,
                              s * C:(s + 1) * C]
                acc_ref[...] += jnp.dot(patch, w_ref[3 * ky + kx],
                                        preferred_element_type=jnp.float32)
        o_ref[s] = jnp.maximum(acc_ref[...], 0.0).astype(jnp.bfloat16)


def _upconv_conv_tanh_kernel(x_ref, a_ref, s_ref, w_ref, b_ref, wp_ref,
                             b3_ref, o_ref, vh_ref, u_ref, acc_ref, p_ref,
                             zi_ref, z_ref, *, H2p, W, C, W2, HW2, ROWS):
    OC = acc_ref.shape[1]
    _sep_upconv_relu(x_ref, a_ref, s_ref, w_ref, b_ref, vh_ref, u_ref,
                     acc_ref, H2p=H2p, W=W, C=C, W2=W2, HW2=HW2, ROWS=ROWS)
    p_ref[0:128, :] = jnp.zeros((128, OC), jnp.bfloat16)
    p_ref[128:128 + HW2, :] = acc_ref[...].astype(jnp.bfloat16)
    p_ref[128 + HW2:, :] = jnp.zeros((128, OC), jnp.bfloat16)
    ncols = jax.lax.broadcasted_iota(jnp.int32, (ROWS, OC), 0) % W2
    for kx in range(3):
        start = 128 - W2 - 1 + kx
        shifted = p_ref[start: start + ROWS, :]
        if kx == 0:
            shifted = jnp.where(ncols == 0, jnp.bfloat16(0), shifted)
        elif kx == 2:
            shifted = jnp.where(ncols == W2 - 1, jnp.bfloat16(0), shifted)
        zi_ref[:, kx * OC:(kx + 1) * OC] = shifted
    z_ref[...] = jnp.dot(zi_ref[...], wp_ref[...],
                         preferred_element_type=jnp.float32)
    o3 = jnp.broadcast_to(b3_ref[...], (HW2, 128))
    for ky in range(3):
        o3 = o3 + z_ref[ky * W2: ky * W2 + HW2, 128 * ky: 128 * ky + 128]
    o_ref[0] = jnp.tanh(jnp.transpose(o3[:, 0:3], (1, 0)))


def _sep_upconv_relu(x_ref, a_ref, s_ref, w_ref, b_ref, vh_ref, u_ref,
                     acc_ref, *, H2p, W, C, W2, HW2, ROWS):
    OC = acc_ref.shape[1]
    vh2 = jnp.dot(a_ref[...], x_ref[0],
                  preferred_element_type=jnp.float32)
    vh_ref[...] = jnp.reshape(vh2, (H2p, W, C)).astype(jnp.bfloat16)
    for kx in range(3):
        u3 = jax.lax.dot_general(
            s_ref[kx], vh_ref[...],
            dimension_numbers=(((2,), (1,)), ((0,), (0,))),
            preferred_element_type=jnp.float32)
        u_ref[kx * ROWS:(kx + 1) * ROWS, :] = (
            jnp.reshape(u3, (ROWS, C)).astype(jnp.bfloat16))
    for kx in range(3):
        base = kx * ROWS
        for ky in range(3):
            patch = u_ref[base + ky * W2: base + ky * W2 + HW2, :]
            d = jnp.dot(patch, w_ref[3 * ky + kx],
                        preferred_element_type=jnp.float32)
            if kx == 0 and ky == 0:
                acc_ref[...] = d + b_ref[...]
            else:
                acc_ref[...] += d
    acc_ref[...] = jnp.maximum(acc_ref[...], 0.0)


def _up_conv(x_flat, H, W, weight, bias):
    N, HW, C = x_flat.shape
    OC = weight.shape[-1]
    H2, W2 = 2 * H, 2 * W
    HW2 = H2 * W2
    ROWS = (H2 + 2) * W2

    m = jnp.asarray(_upsample_mats_np(H, W)).astype(jnp.bfloat16)
    w_taps = weight.reshape(9, C, OC).astype(jnp.bfloat16)
    b_row = bias.reshape(1, OC)

    _kfn = functools.partial(_upconv_kernel, W2=W2, HW2=HW2, ROWS=ROWS)
    return pl.pallas_call(
        _kfn,
        out_shape=jax.ShapeDtypeStruct((N, HW2, OC), jnp.bfloat16),
        grid=(N // 2,),
        in_specs=[
            pl.BlockSpec((2, HW, C), lambda n: (n, 0, 0)),
            pl.BlockSpec((3 * ROWS, HW), lambda n: (0, 0)),
            pl.BlockSpec((9, C, OC), lambda n: (0, 0, 0)),
            pl.BlockSpec((1, OC), lambda n: (0, 0)),
        ],
        out_specs=pl.BlockSpec((2, HW2, OC), lambda n: (n, 0, 0)),
        scratch_shapes=[pltpu.VMEM((3 * ROWS, 2 * C), jnp.bfloat16),
                        pltpu.VMEM((HW2, OC), jnp.float32)],
        compiler_params=pltpu.CompilerParams(
            dimension_semantics=("parallel",),
            vmem_limit_bytes=100 * 1024 * 1024,
        ),
    )(x_flat, m, w_taps, b_row)


def _up_conv_tanh_fused(x_flat, H, W, w2, b2, w3, b3):
    N, HW, C = x_flat.shape
    OC = w2.shape[-1]
    OC3 = w3.shape[-1]
    H2, W2 = 2 * H, 2 * W
    H2p = H2 + 2
    HW2 = H2 * W2
    ROWS = H2p * W2

    ah = _bilinear_matrix_np(H, H2)
    ahp = np.zeros((H2p, H), np.float32)
    ahp[1:H2 + 1] = ah
    aw = _bilinear_matrix_np(W, W2)
    awp = np.zeros((W2 + 2, W), np.float32)
    awp[1:W2 + 1] = aw
    s_np = np.stack([np.broadcast_to(awp[kx:kx + W2], (H2p, W2, W))
                     for kx in range(3)])

    a_mat = jnp.asarray(ahp).astype(jnp.bfloat16)
    s_mat = jnp.asarray(s_np).astype(jnp.bfloat16)
    w_taps = w2.reshape(9, C, OC).astype(jnp.bfloat16)
    b_row = b2.reshape(1, OC)
    wp = jnp.concatenate(
        [jnp.pad(w3[ky].reshape(3 * OC, OC3), ((0, 0), (0, 128 - OC3)))
         for ky in range(3)], axis=-1).astype(jnp.bfloat16)
    b3_row = jnp.pad(b3.reshape(1, OC3), ((0, 0), (0, 128 - OC3)))

    _kfn = functools.partial(_upconv_conv_tanh_kernel, H2p=H2p, W=W, C=C,
                             W2=W2, HW2=HW2, ROWS=ROWS)
    return pl.pallas_call(
        _kfn,
        out_shape=jax.ShapeDtypeStruct((N, OC3, HW2), jnp.float32),
        grid=(N,),
        in_specs=[
            pl.BlockSpec((1, H, W * C), lambda n: (n, 0, 0)),
            pl.BlockSpec((H2p, H), lambda n: (0, 0)),
            pl.BlockSpec((3, H2p, W2, W), lambda n: (0, 0, 0, 0)),
            pl.BlockSpec((9, C, OC), lambda n: (0, 0, 0)),
            pl.BlockSpec((1, OC), lambda n: (0, 0)),
            pl.BlockSpec((3 * OC, 384), lambda n: (0, 0)),
            pl.BlockSpec((1, 128), lambda n: (0, 0)),
        ],
        out_specs=pl.BlockSpec((1, OC3, HW2), lambda n: (n, 0, 0)),
        scratch_shapes=[pltpu.VMEM((H2p, W, C), jnp.bfloat16),
                        pltpu.VMEM((3 * ROWS, C), jnp.bfloat16),
                        pltpu.VMEM((HW2, OC), jnp.float32),
                        pltpu.VMEM((HW2 + 256, OC), jnp.bfloat16),
                        pltpu.VMEM((ROWS, 3 * OC), jnp.bfloat16),
                        pltpu.VMEM((ROWS, 384), jnp.float32)],
        compiler_params=pltpu.CompilerParams(
            dimension_semantics=("parallel",),
            vmem_limit_bytes=100 * 1024 * 1024,
        ),
    )(x_flat.reshape(N, H, W * C), a_mat, s_mat, w_taps, b_row, wp, b3_row)


def kernel(x, w1, b1, w2, b2, w3, b3):
    N, C, H, W = x.shape
    x_flat = jnp.transpose(x, (0, 2, 3, 1)).reshape(N, H * W, C)
    y = _up_conv(x_flat.astype(jnp.bfloat16), H, W, w1, b1)
    y = _up_conv_tanh_fused(y, 2 * H, 2 * W, w2, b2, w3, b3)
    return y.reshape(N, 3, 4 * H, 4 * W)
